# baseline (speedup 1.0000x reference)
import functools

import jax
import jax.numpy as jnp
from jax import lax
from jax.experimental import pallas as pl
from jax.experimental.pallas import tpu as pltpu
from jax.experimental.pallas import tpu_sc as plsc

_NC = 2
_NS = 16
_NW = _NC * _NS
_W = 128
_NBUF = 5
_K = 2


def kernel(tokens, weight):
    B, T = tokens.shape
    V, D = weight.shape
    N = B * T
    steps = N // (_NW * _W)
    idx = tokens.reshape(_NW, steps, _W)
    per_w = steps * _W
    mesh = plsc.VectorSubcoreMesh(core_axis_name="c", subcore_axis_name="s")

    @functools.partial(
        pl.kernel,
        out_type=jax.ShapeDtypeStruct((N, D), weight.dtype),
        mesh=mesh,
        scratch_types=[
            pltpu.VMEM((steps, _W), jnp.int32),
            pltpu.VMEM((_NBUF, _W, D), jnp.float32),
            pltpu.SemaphoreType.DMA((_NBUF,)),
            pltpu.SemaphoreType.DMA((_NBUF,)),
        ],
    )
    def gather_kernel(w_hbm, i_hbm, o_hbm, idx_v, rows_v, gsem, osem):
        wid = lax.axis_index("s") * _NC + lax.axis_index("c")
        base = wid * per_w
        pltpu.sync_copy(i_hbm.at[wid], idx_v)

        def gather(t, slot):
            return pltpu.make_async_copy(
                w_hbm.at[idx_v.at[t]], rows_v.at[slot], gsem.at[slot]
            )

        def drain(t, slot):
            return pltpu.make_async_copy(
                rows_v.at[slot], o_hbm.at[pl.ds(base + t * _W, _W)], osem.at[slot]
            )

        for t in range(_K):
            gather(t, t).start()

        @pl.loop(0, steps, step=_NBUF)
        def _(c0):
            for b in range(_NBUF):
                c = c0 + b
                gather(c, b).wait()
                drain(c, b).start()
                t2 = c + _K
                s2 = (b + _K) % _NBUF

                @pl.when(t2 < steps)
                def _():
                    @pl.when(t2 >= _NBUF)
                    def _():
                        drain(t2 - _NBUF, s2).wait()

                    gather(t2, s2).start()

        for b in range(_NBUF):
            drain(steps - _NBUF + b, b).wait()

    out = gather_kernel(weight, idx)
    return out.reshape(B, T, D)


# speedup vs baseline: 1.0063x; 1.0063x over previous
import functools

import jax
import jax.numpy as jnp
from jax import lax
from jax.experimental import pallas as pl
from jax.experimental.pallas import tpu as pltpu
from jax.experimental.pallas import tpu_sc as plsc

_NC = 2
_NS = 16
_NW = _NC * _NS
_W = 128
_NBUF = 5
_K = 4


def kernel(tokens, weight):
    B, T = tokens.shape
    V, D = weight.shape
    N = B * T
    steps = N // (_NW * _W)
    idx = tokens.reshape(_NW, steps, _W)
    per_w = steps * _W
    mesh = plsc.VectorSubcoreMesh(core_axis_name="c", subcore_axis_name="s")

    @functools.partial(
        pl.kernel,
        out_type=jax.ShapeDtypeStruct((N, D), weight.dtype),
        mesh=mesh,
        scratch_types=[
            pltpu.VMEM((steps, _W), jnp.int32),
            pltpu.VMEM((_NBUF, _W, D), jnp.float32),
            pltpu.SemaphoreType.DMA((_NBUF,)),
            pltpu.SemaphoreType.DMA((_NBUF,)),
        ],
    )
    def gather_kernel(w_hbm, i_hbm, o_hbm, idx_v, rows_v, gsem, osem):
        wid = lax.axis_index("s") * _NC + lax.axis_index("c")
        base = wid * per_w
        pltpu.sync_copy(i_hbm.at[wid], idx_v)

        def gather(t, slot):
            return pltpu.make_async_copy(
                w_hbm.at[idx_v.at[t]], rows_v.at[slot], gsem.at[slot]
            )

        def drain(t, slot):
            return pltpu.make_async_copy(
                rows_v.at[slot], o_hbm.at[pl.ds(base + t * _W, _W)], osem.at[slot]
            )

        for t in range(_K):
            gather(t, t).start()

        @pl.loop(0, steps, step=_NBUF)
        def _(c0):
            for b in range(_NBUF):
                c = c0 + b
                gather(c, b).wait()
                drain(c, b).start()
                t2 = c + _K
                s2 = (b + _K) % _NBUF

                @pl.when(t2 < steps)
                def _():
                    @pl.when(t2 >= _NBUF)
                    def _():
                        drain(t2 - _NBUF, s2).wait()

                    gather(t2, s2).start()

        for b in range(_NBUF):
            drain(steps - _NBUF + b, b).wait()

    out = gather_kernel(weight, idx)
    return out.reshape(B, T, D)


# speedup vs baseline: 1.0103x; 1.0040x over previous
import functools

import jax
import jax.numpy as jnp
from jax import lax
from jax.experimental import pallas as pl
from jax.experimental.pallas import tpu as pltpu
from jax.experimental.pallas import tpu_sc as plsc

_NC = 2
_NS = 16
_NW = _NC * _NS
_W = 128
_NBUF = 5
_K = 3


def kernel(tokens, weight):
    B, T = tokens.shape
    V, D = weight.shape
    N = B * T
    steps = N // (_NW * _W)
    idx = tokens.reshape(_NW, steps, _W)
    per_w = steps * _W
    mesh = plsc.VectorSubcoreMesh(core_axis_name="c", subcore_axis_name="s")

    @functools.partial(
        pl.kernel,
        out_type=jax.ShapeDtypeStruct((N, D), weight.dtype),
        mesh=mesh,
        scratch_types=[
            pltpu.VMEM((steps, _W), jnp.int32),
            pltpu.VMEM((_NBUF, _W, D), jnp.float32),
            pltpu.SemaphoreType.DMA((_NBUF,)),
            pltpu.SemaphoreType.DMA((_NBUF,)),
        ],
    )
    def gather_kernel(w_hbm, i_hbm, o_hbm, idx_v, rows_v, gsem, osem):
        wid = lax.axis_index("s") * _NC + lax.axis_index("c")
        base = wid * per_w
        pltpu.sync_copy(i_hbm.at[wid], idx_v)

        def gather(t, slot):
            return pltpu.make_async_copy(
                w_hbm.at[idx_v.at[t]], rows_v.at[slot], gsem.at[slot]
            )

        def drain(t, slot):
            return pltpu.make_async_copy(
                rows_v.at[slot], o_hbm.at[pl.ds(base + t * _W, _W)], osem.at[slot]
            )

        for t in range(_K):
            gather(t, t).start()

        @pl.loop(0, steps, step=_NBUF)
        def _(c0):
            for b in range(_NBUF):
                c = c0 + b
                gather(c, b).wait()
                drain(c, b).start()
                t2 = c + _K
                s2 = (b + _K) % _NBUF

                @pl.when(t2 < steps)
                def _():
                    @pl.when(t2 >= _NBUF)
                    def _():
                        drain(t2 - _NBUF, s2).wait()

                    gather(t2, s2).start()

        for b in range(_NBUF):
            drain(steps - _NBUF + b, b).wait()

    out = gather_kernel(weight, idx)
    return out.reshape(B, T, D)
